# Initial kernel scaffold; baseline (speedup 1.0000x reference)
import jax
import jax.numpy as jnp
from jax.experimental import pallas as pl


def kernel(x, coarse_coord, W1, b1, W2, b2, Ws, bs, Wm, bm, Wv, bv):
    raise NotImplementedError("write your pallas kernel here")


# speedup vs baseline: 2.8241x; 2.8241x over previous
import functools

import jax
import jax.numpy as jnp
from jax.experimental import pallas as pl


def _moe_kernel(x_ref, c_ref, W1_ref, b1_ref, W2_ref, b2_ref,
                Wall_ref, bsf_ref, Wbd_ref, bmv_ref, Csum_ref,
                refined_ref, delta_ref, loss_ref, *, n_tokens, n_experts):
    T = x_ref.shape[0]
    E = n_experts
    x = x_ref[...]

    h1 = jnp.maximum(jnp.dot(x, W1_ref[...],
                             preferred_element_type=jnp.float32) + b1_ref[...], 0.0)
    logits = jnp.dot(h1, W2_ref[...],
                     preferred_element_type=jnp.float32) + b2_ref[...]
    z = logits - jnp.max(logits, axis=-1, keepdims=True)
    ez = jnp.exp(z)
    p = ez / jnp.sum(ez, axis=-1, keepdims=True)

    ent = -jnp.sum(p * jnp.log(p + 1e-8), axis=-1)
    part = jnp.sum(ent).reshape(1, 1) / n_tokens

    @pl.when(pl.program_id(0) == 0)
    def _():
        loss_ref[...] = jnp.zeros((1, 1), jnp.float32)

    loss_ref[...] += part

    e_iota = jax.lax.broadcasted_iota(jnp.int32, (T, E), 1)
    v1 = jnp.max(p, axis=-1, keepdims=True)
    i1 = jnp.min(jnp.where(p >= v1, e_iota, E), axis=-1, keepdims=True)
    p2 = jnp.where(e_iota == i1, -1.0, p)
    v2 = jnp.max(p2, axis=-1, keepdims=True)
    i2 = jnp.min(jnp.where(p2 >= v2, e_iota, E), axis=-1, keepdims=True)
    denom = v1 + v2
    w1 = v1 / denom
    w2 = v2 / denom

    feats = jnp.maximum(jnp.dot(x, Wall_ref[...],
                                preferred_element_type=jnp.float32) + bsf_ref[...], 0.0)
    outs = jnp.dot(feats, Wbd_ref[...],
                   preferred_element_type=jnp.float32) + bmv_ref[...]
    EC = E * 4
    c_iota = jax.lax.broadcasted_iota(jnp.int32, (T, EC), 1)
    is_mean = (c_iota % 4) < 2
    sp = jnp.maximum(outs, 0.0) + jnp.log1p(jnp.exp(-jnp.abs(outs)))
    acts = jnp.where(is_mean, jnp.tanh(outs), sp)

    e_of_c = c_iota // 4
    wexp = (jnp.where(e_of_c == i1, w1, 0.0)
            + jnp.where(e_of_c == i2, w2, 0.0))
    delta = jnp.dot(acts * wexp, Csum_ref[...],
                    preferred_element_type=jnp.float32)
    delta_ref[...] = delta
    refined_ref[...] = jnp.clip(c_ref[...] + delta[:, :2] * 0.002, 0.0, 1.0)


def kernel(x, coarse_coord, W1, b1, W2, b2, Ws, bs, Wm, bm, Wv, bv):
    B, S, D = x.shape
    GH = W1.shape[1]
    E = W2.shape[1]
    H = Ws.shape[2]
    N = B * S
    T = 512
    grid = N // T

    x2 = x.reshape(N, D)
    c2 = coarse_coord.reshape(N, 2)
    Wall = jnp.transpose(Ws, (1, 0, 2)).reshape(D, E * H)
    bsf = bs.reshape(1, E * H)
    Wmv = jnp.concatenate([Wm, Wv], axis=-1)
    eye = jnp.eye(E, dtype=x.dtype)
    Wbd = (eye[:, None, :, None] * Wmv[:, :, None, :]).reshape(E * H, E * 4)
    bmvf = jnp.concatenate([bm, bv], axis=-1).reshape(1, E * 4)
    Csum = jnp.tile(jnp.eye(4, dtype=x.dtype), (E, 1))
    b1r = b1.reshape(1, GH)
    b2r = b2.reshape(1, E)

    body = functools.partial(_moe_kernel, n_tokens=float(N), n_experts=E)
    full = lambda i: (0, 0)
    refined, delta, loss = pl.pallas_call(
        body,
        grid=(grid,),
        in_specs=[
            pl.BlockSpec((T, D), lambda i: (i, 0)),
            pl.BlockSpec((T, 2), lambda i: (i, 0)),
            pl.BlockSpec((D, GH), full),
            pl.BlockSpec((1, GH), full),
            pl.BlockSpec((GH, E), full),
            pl.BlockSpec((1, E), full),
            pl.BlockSpec((D, E * H), full),
            pl.BlockSpec((1, E * H), full),
            pl.BlockSpec((E * H, E * 4), full),
            pl.BlockSpec((1, E * 4), full),
            pl.BlockSpec((E * 4, 4), full),
        ],
        out_specs=[
            pl.BlockSpec((T, 2), lambda i: (i, 0)),
            pl.BlockSpec((T, 4), lambda i: (i, 0)),
            pl.BlockSpec((1, 1), full),
        ],
        out_shape=[
            jax.ShapeDtypeStruct((N, 2), jnp.float32),
            jax.ShapeDtypeStruct((N, 4), jnp.float32),
            jax.ShapeDtypeStruct((1, 1), jnp.float32),
        ],
    )(x2, c2, W1, b1r, W2, b2r, Wall, bsf, Wbd, bmvf, Csum)

    return (refined.reshape(B, S, 2), loss[0, 0], delta.reshape(B, S, 4))


# speedup vs baseline: 2.8431x; 1.0068x over previous
import functools

import jax
import jax.numpy as jnp
from jax.experimental import pallas as pl


def _moe_kernel(x_ref, c_ref, W1_ref, b1_ref, W2_ref, b2_ref,
                Wall_ref, bsf_ref, Wbd_ref, bmv_ref, Csum_ref,
                refined_ref, delta_ref, loss_ref, *, n_tokens, n_experts):
    T = x_ref.shape[0]
    E = n_experts
    x = x_ref[...]

    h1 = jnp.maximum(jnp.dot(x, W1_ref[...],
                             preferred_element_type=jnp.float32) + b1_ref[...], 0.0)
    logits = jnp.dot(h1, W2_ref[...],
                     preferred_element_type=jnp.float32) + b2_ref[...]
    z = logits - jnp.max(logits, axis=-1, keepdims=True)
    ez = jnp.exp(z)
    p = ez / jnp.sum(ez, axis=-1, keepdims=True)

    ent = -jnp.sum(p * jnp.log(p + 1e-8), axis=-1)
    part = jnp.sum(ent).reshape(1, 1) / n_tokens

    @pl.when(pl.program_id(0) == 0)
    def _():
        loss_ref[...] = jnp.zeros((1, 1), jnp.float32)

    loss_ref[...] += part

    e_iota = jax.lax.broadcasted_iota(jnp.int32, (T, E), 1)
    v1 = jnp.max(p, axis=-1, keepdims=True)
    i1 = jnp.min(jnp.where(p >= v1, e_iota, E), axis=-1, keepdims=True)
    p2 = jnp.where(e_iota == i1, -1.0, p)
    v2 = jnp.max(p2, axis=-1, keepdims=True)
    i2 = jnp.min(jnp.where(p2 >= v2, e_iota, E), axis=-1, keepdims=True)
    denom = v1 + v2
    w1 = v1 / denom
    w2 = v2 / denom

    feats = jnp.maximum(jnp.dot(x.astype(jnp.bfloat16), Wall_ref[...],
                                preferred_element_type=jnp.float32) + bsf_ref[...], 0.0)
    outs = jnp.dot(feats.astype(jnp.bfloat16), Wbd_ref[...],
                   preferred_element_type=jnp.float32) + bmv_ref[...]
    EC = E * 4
    c_iota = jax.lax.broadcasted_iota(jnp.int32, (T, EC), 1)
    is_mean = (c_iota % 4) < 2
    sp = jnp.maximum(outs, 0.0) + jnp.log1p(jnp.exp(-jnp.abs(outs)))
    acts = jnp.where(is_mean, jnp.tanh(outs), sp)

    e_of_c = c_iota // 4
    wexp = (jnp.where(e_of_c == i1, w1, 0.0)
            + jnp.where(e_of_c == i2, w2, 0.0))
    delta = jnp.dot(acts * wexp, Csum_ref[...],
                    preferred_element_type=jnp.float32)
    delta_ref[...] = delta
    refined_ref[...] = jnp.clip(c_ref[...] + delta[:, :2] * 0.002, 0.0, 1.0)


def kernel(x, coarse_coord, W1, b1, W2, b2, Ws, bs, Wm, bm, Wv, bv):
    B, S, D = x.shape
    GH = W1.shape[1]
    E = W2.shape[1]
    H = Ws.shape[2]
    N = B * S
    T = 512
    grid = N // T

    x2 = x.reshape(N, D)
    c2 = coarse_coord.reshape(N, 2)
    Wall = jnp.transpose(Ws, (1, 0, 2)).reshape(D, E * H).astype(jnp.bfloat16)
    bsf = bs.reshape(1, E * H)
    Wmv = jnp.concatenate([Wm, Wv], axis=-1)
    eye = jnp.eye(E, dtype=x.dtype)
    Wbd = (eye[:, None, :, None] * Wmv[:, :, None, :]).reshape(E * H, E * 4).astype(jnp.bfloat16)
    bmvf = jnp.concatenate([bm, bv], axis=-1).reshape(1, E * 4)
    Csum = jnp.tile(jnp.eye(4, dtype=x.dtype), (E, 1))
    b1r = b1.reshape(1, GH)
    b2r = b2.reshape(1, E)

    body = functools.partial(_moe_kernel, n_tokens=float(N), n_experts=E)
    full = lambda i: (0, 0)
    refined, delta, loss = pl.pallas_call(
        body,
        grid=(grid,),
        in_specs=[
            pl.BlockSpec((T, D), lambda i: (i, 0)),
            pl.BlockSpec((T, 2), lambda i: (i, 0)),
            pl.BlockSpec((D, GH), full),
            pl.BlockSpec((1, GH), full),
            pl.BlockSpec((GH, E), full),
            pl.BlockSpec((1, E), full),
            pl.BlockSpec((D, E * H), full),
            pl.BlockSpec((1, E * H), full),
            pl.BlockSpec((E * H, E * 4), full),
            pl.BlockSpec((1, E * 4), full),
            pl.BlockSpec((E * 4, 4), full),
        ],
        out_specs=[
            pl.BlockSpec((T, 2), lambda i: (i, 0)),
            pl.BlockSpec((T, 4), lambda i: (i, 0)),
            pl.BlockSpec((1, 1), full),
        ],
        out_shape=[
            jax.ShapeDtypeStruct((N, 2), jnp.float32),
            jax.ShapeDtypeStruct((N, 4), jnp.float32),
            jax.ShapeDtypeStruct((1, 1), jnp.float32),
        ],
    )(x2, c2, W1, b1r, W2, b2r, Wall, bsf, Wbd, bmvf, Csum)

    return (refined.reshape(B, S, 2), loss[0, 0], delta.reshape(B, S, 4))


# speedup vs baseline: 2.8929x; 1.0175x over previous
import functools

import jax
import jax.numpy as jnp
from jax.experimental import pallas as pl


def _moe_kernel(x_ref, c_ref, W1_ref, b1_ref, W2_ref, b2_ref,
                Wall_ref, bsf_ref, Wbd_ref, bmv_ref, Rep_ref, Csum_ref,
                refined_ref, delta_ref, loss_ref, *, n_tokens):
    x = x_ref[...]

    h1 = jnp.maximum(jnp.dot(x, W1_ref[...],
                             preferred_element_type=jnp.float32) + b1_ref[...], 0.0)
    zT = jax.lax.dot_general(W2_ref[...], h1, (((0,), (1,)), ((), ())),
                             preferred_element_type=jnp.float32) + b2_ref[...]
    zT = zT - jnp.max(zT, axis=0, keepdims=True)
    ez = jnp.exp(zT)
    S = jnp.sum(ez, axis=0, keepdims=True)

    ent = jnp.log(S) - jnp.sum(ez * zT, axis=0, keepdims=True) / S
    part = (jnp.sum(ent) / n_tokens).reshape(1, 1)

    @pl.when(pl.program_id(0) == 0)
    def _():
        loss_ref[...] = jnp.zeros((1, 1), jnp.float32)

    loss_ref[...] += part

    v1 = jnp.max(ez, axis=0, keepdims=True)
    m1 = ez >= v1
    c1 = jnp.sum(m1.astype(jnp.float32), axis=0, keepdims=True)
    vr = jnp.max(jnp.where(m1, -1.0, ez), axis=0, keepdims=True)
    v2 = jnp.where(c1 > 1.0, v1, vr)
    wT = jnp.where(ez >= v2, ez, 0.0) / (v1 + v2)

    feats = jnp.maximum(jnp.dot(x.astype(jnp.bfloat16), Wall_ref[...],
                                preferred_element_type=jnp.float32) + bsf_ref[...], 0.0)
    outs = jnp.dot(feats.astype(jnp.bfloat16), Wbd_ref[...],
                   preferred_element_type=jnp.float32) + bmv_ref[...]
    T = x.shape[0]
    EC = outs.shape[1]
    c_iota = jax.lax.broadcasted_iota(jnp.int32, (T, EC), 1)
    is_mean = (c_iota % 4) < 2
    sp = jnp.maximum(outs, 0.0) + jnp.log1p(jnp.exp(-jnp.abs(outs)))
    acts = jnp.where(is_mean, jnp.tanh(outs), sp)

    w32 = jax.lax.dot_general(wT, Rep_ref[...], (((0,), (0,)), ((), ())),
                              preferred_element_type=jnp.float32)
    delta = jnp.dot(acts * w32, Csum_ref[...],
                    preferred_element_type=jnp.float32)
    delta_ref[...] = delta
    refined_ref[...] = jnp.clip(c_ref[...] + delta[:, :2] * 0.002, 0.0, 1.0)


def kernel(x, coarse_coord, W1, b1, W2, b2, Ws, bs, Wm, bm, Wv, bv):
    B, S, D = x.shape
    GH = W1.shape[1]
    E = W2.shape[1]
    H = Ws.shape[2]
    N = B * S
    T = 512
    grid = N // T

    x2 = x.reshape(N, D)
    c2 = coarse_coord.reshape(N, 2)
    Wall = jnp.transpose(Ws, (1, 0, 2)).reshape(D, E * H).astype(jnp.bfloat16)
    bsf = bs.reshape(1, E * H)
    Wmv = jnp.concatenate([Wm, Wv], axis=-1)
    eye = jnp.eye(E, dtype=x.dtype)
    Wbd = (eye[:, None, :, None] * Wmv[:, :, None, :]).reshape(E * H, E * 4).astype(jnp.bfloat16)
    bmvf = jnp.concatenate([bm, bv], axis=-1).reshape(1, E * 4)
    Rep = jnp.kron(jnp.eye(E, dtype=x.dtype), jnp.ones((1, 4), x.dtype))
    Csum = jnp.tile(jnp.eye(4, dtype=x.dtype), (E, 1))
    b1r = b1.reshape(1, GH)
    b2c = b2.reshape(E, 1)

    body = functools.partial(_moe_kernel, n_tokens=float(N))
    full = lambda i: (0, 0)
    refined, delta, loss = pl.pallas_call(
        body,
        grid=(grid,),
        in_specs=[
            pl.BlockSpec((T, D), lambda i: (i, 0)),
            pl.BlockSpec((T, 2), lambda i: (i, 0)),
            pl.BlockSpec((D, GH), full),
            pl.BlockSpec((1, GH), full),
            pl.BlockSpec((GH, E), full),
            pl.BlockSpec((E, 1), full),
            pl.BlockSpec((D, E * H), full),
            pl.BlockSpec((1, E * H), full),
            pl.BlockSpec((E * H, E * 4), full),
            pl.BlockSpec((1, E * 4), full),
            pl.BlockSpec((E, E * 4), full),
            pl.BlockSpec((E * 4, 4), full),
        ],
        out_specs=[
            pl.BlockSpec((T, 2), lambda i: (i, 0)),
            pl.BlockSpec((T, 4), lambda i: (i, 0)),
            pl.BlockSpec((1, 1), full),
        ],
        out_shape=[
            jax.ShapeDtypeStruct((N, 2), jnp.float32),
            jax.ShapeDtypeStruct((N, 4), jnp.float32),
            jax.ShapeDtypeStruct((1, 1), jnp.float32),
        ],
    )(x2, c2, W1, b1r, W2, b2c, Wall, bsf, Wbd, bmvf, Rep, Csum)

    return (refined.reshape(B, S, 2), loss[0, 0], delta.reshape(B, S, 4))


# speedup vs baseline: 2.9253x; 1.0112x over previous
import functools

import jax
import jax.numpy as jnp
from jax.experimental import pallas as pl


def _moe_kernel(x_ref, c_ref, W1h_ref, b1_ref, W2_ref, b2_ref,
                Wbig_ref, bsf_ref, Wbd_ref, bmv_ref, Rep_ref, Csum_ref,
                refined_ref, delta_ref, loss_ref, *, n_tokens, eh):
    x = x_ref[...]
    xh = x.astype(jnp.bfloat16)
    xl = (x - xh.astype(jnp.float32)).astype(jnp.bfloat16)

    big = jnp.dot(xh, Wbig_ref[...], preferred_element_type=jnp.float32)
    GH = W1h_ref.shape[1]
    feats = jnp.maximum(big[:, :eh] + bsf_ref[...], 0.0)

    h1 = jnp.maximum(
        big[:, eh:eh + GH] + big[:, eh + GH:eh + 2 * GH]
        + jnp.dot(xl, W1h_ref[...], preferred_element_type=jnp.float32)
        + b1_ref[...], 0.0)
    zT = jax.lax.dot_general(W2_ref[...], h1, (((0,), (1,)), ((), ())),
                             preferred_element_type=jnp.float32) + b2_ref[...]
    zT = zT - jnp.max(zT, axis=0, keepdims=True)
    ez = jnp.exp(zT)
    S = jnp.sum(ez, axis=0, keepdims=True)

    ent = jnp.log(S) - jnp.sum(ez * zT, axis=0, keepdims=True) / S
    part = (jnp.sum(ent) / n_tokens).reshape(1, 1)

    @pl.when(pl.program_id(0) == 0)
    def _():
        loss_ref[...] = jnp.zeros((1, 1), jnp.float32)

    loss_ref[...] += part

    v1 = jnp.max(ez, axis=0, keepdims=True)
    m1 = ez >= v1
    c1 = jnp.sum(m1.astype(jnp.float32), axis=0, keepdims=True)
    vr = jnp.max(jnp.where(m1, -1.0, ez), axis=0, keepdims=True)
    v2 = jnp.where(c1 > 1.0, v1, vr)
    wT = jnp.where(ez >= v2, ez, 0.0) / (v1 + v2)

    outs = jnp.dot(feats.astype(jnp.bfloat16), Wbd_ref[...],
                   preferred_element_type=jnp.float32) + bmv_ref[...]
    T = x.shape[0]
    EC = outs.shape[1]
    c_iota = jax.lax.broadcasted_iota(jnp.int32, (T, EC), 1)
    is_mean = (c_iota % 4) < 2
    sp = jnp.maximum(outs, 0.0) + jnp.log1p(jnp.exp(-jnp.abs(outs)))
    acts = jnp.where(is_mean, jnp.tanh(outs), sp)

    w32 = jax.lax.dot_general(wT, Rep_ref[...], (((0,), (0,)), ((), ())),
                              preferred_element_type=jnp.float32)
    delta = jnp.dot(acts * w32, Csum_ref[...],
                    preferred_element_type=jnp.float32)
    delta_ref[...] = delta
    refined_ref[...] = jnp.clip(c_ref[...] + delta[:, :2] * 0.002, 0.0, 1.0)


def kernel(x, coarse_coord, W1, b1, W2, b2, Ws, bs, Wm, bm, Wv, bv):
    B, S, D = x.shape
    GH = W1.shape[1]
    E = W2.shape[1]
    H = Ws.shape[2]
    N = B * S
    T = 1024
    grid = N // T
    EH = E * H

    x2 = x.reshape(N, D)
    c2 = coarse_coord.reshape(N, 2)
    Wall = jnp.transpose(Ws, (1, 0, 2)).reshape(D, EH)
    W1h = W1.astype(jnp.bfloat16)
    W1l = (W1 - W1h.astype(jnp.float32)).astype(jnp.bfloat16)
    Wbig = jnp.concatenate(
        [Wall.astype(jnp.bfloat16), W1h, W1l], axis=1)
    bsf = bs.reshape(1, EH)
    Wmv = jnp.concatenate([Wm, Wv], axis=-1)
    eye = jnp.eye(E, dtype=x.dtype)
    Wbd = (eye[:, None, :, None] * Wmv[:, :, None, :]).reshape(EH, E * 4).astype(jnp.bfloat16)
    bmvf = jnp.concatenate([bm, bv], axis=-1).reshape(1, E * 4)
    Rep = jnp.kron(jnp.eye(E, dtype=x.dtype), jnp.ones((1, 4), x.dtype))
    Csum = jnp.tile(jnp.eye(4, dtype=x.dtype), (E, 1))
    b1r = b1.reshape(1, GH)
    b2c = b2.reshape(E, 1)

    body = functools.partial(_moe_kernel, n_tokens=float(N), eh=EH)
    full = lambda i: (0, 0)
    refined, delta, loss = pl.pallas_call(
        body,
        grid=(grid,),
        in_specs=[
            pl.BlockSpec((T, D), lambda i: (i, 0)),
            pl.BlockSpec((T, 2), lambda i: (i, 0)),
            pl.BlockSpec((D, GH), full),
            pl.BlockSpec((1, GH), full),
            pl.BlockSpec((GH, E), full),
            pl.BlockSpec((E, 1), full),
            pl.BlockSpec((D, EH + 2 * GH), full),
            pl.BlockSpec((1, EH), full),
            pl.BlockSpec((EH, E * 4), full),
            pl.BlockSpec((1, E * 4), full),
            pl.BlockSpec((E, E * 4), full),
            pl.BlockSpec((E * 4, 4), full),
        ],
        out_specs=[
            pl.BlockSpec((T, 2), lambda i: (i, 0)),
            pl.BlockSpec((T, 4), lambda i: (i, 0)),
            pl.BlockSpec((1, 1), full),
        ],
        out_shape=[
            jax.ShapeDtypeStruct((N, 2), jnp.float32),
            jax.ShapeDtypeStruct((N, 4), jnp.float32),
            jax.ShapeDtypeStruct((1, 1), jnp.float32),
        ],
    )(x2, c2, W1h, b1r, W2, b2c, Wbig, bsf, Wbd, bmvf, Rep, Csum)

    return (refined.reshape(B, S, 2), loss[0, 0], delta.reshape(B, S, 4))


# speedup vs baseline: 2.9415x; 1.0055x over previous
import functools

import jax
import jax.numpy as jnp
from jax.experimental import pallas as pl


def _moe_kernel(x_ref, c_ref, W1hl_ref, b1_ref, W2_ref, b2_ref,
                Wbig_ref, bsf_ref, Wbd_ref, bmv_ref, Rep_ref, Csum_ref,
                refined_ref, delta_ref, loss_ref, *, n_tokens, eh):
    x = x_ref[...]
    xh = x.astype(jnp.bfloat16)
    xl = (x - xh.astype(jnp.float32)).astype(jnp.bfloat16)

    big = jnp.dot(xh, Wbig_ref[...], preferred_element_type=jnp.float32)
    GH = W1hl_ref.shape[1] // 2
    feats = jnp.maximum(big[:, :eh] + bsf_ref[...], 0.0)

    lo = jnp.dot(xl, W1hl_ref[...], preferred_element_type=jnp.float32)
    h1 = jnp.maximum(
        big[:, eh:eh + GH] + big[:, eh + GH:eh + 2 * GH]
        + lo[:, :GH] + lo[:, GH:]
        + b1_ref[...], 0.0)
    zT = jax.lax.dot_general(W2_ref[...], h1, (((0,), (1,)), ((), ())),
                             preferred_element_type=jnp.float32) + b2_ref[...]
    zT = zT - jnp.max(zT, axis=0, keepdims=True)
    ez = jnp.exp(zT)
    S = jnp.sum(ez, axis=0, keepdims=True)

    ent = jnp.log(S) - jnp.sum(ez * zT, axis=0, keepdims=True) / S
    part = (jnp.sum(ent) / n_tokens).reshape(1, 1)

    @pl.when(pl.program_id(0) == 0)
    def _():
        loss_ref[...] = jnp.zeros((1, 1), jnp.float32)

    loss_ref[...] += part

    v1 = jnp.max(ez, axis=0, keepdims=True)
    m1 = ez >= v1
    c1 = jnp.sum(m1.astype(jnp.float32), axis=0, keepdims=True)
    vr = jnp.max(jnp.where(m1, -1.0, ez), axis=0, keepdims=True)
    v2 = jnp.where(c1 > 1.0, v1, vr)
    wT = jnp.where(ez >= v2, ez, 0.0) / (v1 + v2)

    outs = jnp.dot(feats.astype(jnp.bfloat16), Wbd_ref[...],
                   preferred_element_type=jnp.float32) + bmv_ref[...]
    T = x.shape[0]
    EC = outs.shape[1]
    c_iota = jax.lax.broadcasted_iota(jnp.int32, (T, EC), 1)
    is_mean = (c_iota % 4) < 2
    sp = jnp.maximum(outs, 0.0) + jnp.log1p(jnp.exp(-jnp.abs(outs)))
    acts = jnp.where(is_mean, jnp.tanh(outs), sp)

    w32 = jax.lax.dot_general(wT, Rep_ref[...], (((0,), (0,)), ((), ())),
                              preferred_element_type=jnp.float32)
    delta = jnp.dot(acts * w32, Csum_ref[...],
                    preferred_element_type=jnp.float32)
    delta_ref[...] = delta
    refined_ref[...] = jnp.clip(c_ref[...] + delta[:, :2] * 0.002, 0.0, 1.0)


def kernel(x, coarse_coord, W1, b1, W2, b2, Ws, bs, Wm, bm, Wv, bv):
    B, S, D = x.shape
    GH = W1.shape[1]
    E = W2.shape[1]
    H = Ws.shape[2]
    N = B * S
    T = 1024
    grid = N // T
    EH = E * H

    x2 = x.reshape(N, D)
    c2 = coarse_coord.reshape(N, 2)
    Wall = jnp.transpose(Ws, (1, 0, 2)).reshape(D, EH)
    W1h = W1.astype(jnp.bfloat16)
    W1l = (W1 - W1h.astype(jnp.float32)).astype(jnp.bfloat16)
    W1hl = jnp.concatenate([W1h, W1l], axis=1)
    Wbig = jnp.concatenate(
        [Wall.astype(jnp.bfloat16), W1h, W1l], axis=1)
    bsf = bs.reshape(1, EH)
    Wmv = jnp.concatenate([Wm, Wv], axis=-1)
    eye = jnp.eye(E, dtype=x.dtype)
    Wbd = (eye[:, None, :, None] * Wmv[:, :, None, :]).reshape(EH, E * 4).astype(jnp.bfloat16)
    bmvf = jnp.concatenate([bm, bv], axis=-1).reshape(1, E * 4)
    Rep = jnp.kron(jnp.eye(E, dtype=x.dtype), jnp.ones((1, 4), x.dtype))
    Csum = jnp.tile(jnp.eye(4, dtype=x.dtype), (E, 1))
    b1r = b1.reshape(1, GH)
    b2c = b2.reshape(E, 1)

    body = functools.partial(_moe_kernel, n_tokens=float(N), eh=EH)
    full = lambda i: (0, 0)
    refined, delta, loss = pl.pallas_call(
        body,
        grid=(grid,),
        in_specs=[
            pl.BlockSpec((T, D), lambda i: (i, 0)),
            pl.BlockSpec((T, 2), lambda i: (i, 0)),
            pl.BlockSpec((D, 2 * GH), full),
            pl.BlockSpec((1, GH), full),
            pl.BlockSpec((GH, E), full),
            pl.BlockSpec((E, 1), full),
            pl.BlockSpec((D, EH + 2 * GH), full),
            pl.BlockSpec((1, EH), full),
            pl.BlockSpec((EH, E * 4), full),
            pl.BlockSpec((1, E * 4), full),
            pl.BlockSpec((E, E * 4), full),
            pl.BlockSpec((E * 4, 4), full),
        ],
        out_specs=[
            pl.BlockSpec((T, 2), lambda i: (i, 0)),
            pl.BlockSpec((T, 4), lambda i: (i, 0)),
            pl.BlockSpec((1, 1), full),
        ],
        out_shape=[
            jax.ShapeDtypeStruct((N, 2), jnp.float32),
            jax.ShapeDtypeStruct((N, 4), jnp.float32),
            jax.ShapeDtypeStruct((1, 1), jnp.float32),
        ],
    )(x2, c2, W1hl, b1r, W2, b2c, Wbig, bsf, Wbd, bmvf, Rep, Csum)

    return (refined.reshape(B, S, 2), loss[0, 0], delta.reshape(B, S, 4))


# speedup vs baseline: 3.1138x; 1.0586x over previous
import functools

import jax
import jax.numpy as jnp
from jax.experimental import pallas as pl


def _moe_kernel(x_ref, c_ref, W1_ref, b1_ref, W2_ref, b2_ref,
                Wbig_ref, bsf_ref, Wbd_ref, bmv_ref, Rep_ref, Csum_ref,
                refined_ref, delta_ref, loss_ref, *, n_tokens, eh):
    x = x_ref[...]

    big = jnp.dot(x.astype(jnp.bfloat16), Wbig_ref[...],
                  preferred_element_type=jnp.float32)
    feats = jnp.maximum(big[:, :eh] + bsf_ref[...], 0.0)

    h1 = jnp.maximum(jnp.dot(x, W1_ref[...],
                             preferred_element_type=jnp.float32) + b1_ref[...], 0.0)
    zT = jax.lax.dot_general(W2_ref[...], h1, (((0,), (1,)), ((), ())),
                             preferred_element_type=jnp.float32) + b2_ref[...]
    zT = zT - jnp.max(zT, axis=0, keepdims=True)
    ez = jnp.exp(zT)
    S = jnp.sum(ez, axis=0, keepdims=True)

    ent = jnp.log(S) - jnp.sum(ez * zT, axis=0, keepdims=True) / S
    part = (jnp.sum(ent) / n_tokens).reshape(1, 1)

    @pl.when(pl.program_id(0) == 0)
    def _():
        loss_ref[...] = jnp.zeros((1, 1), jnp.float32)

    loss_ref[...] += part

    v1 = jnp.max(ez, axis=0, keepdims=True)
    m1 = ez >= v1
    c1 = jnp.sum(m1.astype(jnp.float32), axis=0, keepdims=True)
    vr = jnp.max(jnp.where(m1, -1.0, ez), axis=0, keepdims=True)
    v2 = jnp.where(c1 > 1.0, v1, vr)
    wT = jnp.where(ez >= v2, ez, 0.0) / (v1 + v2)

    outs = jnp.dot(feats.astype(jnp.bfloat16), Wbd_ref[...],
                   preferred_element_type=jnp.float32) + bmv_ref[...]
    T = x.shape[0]
    EC = outs.shape[1]
    c_iota = jax.lax.broadcasted_iota(jnp.int32, (T, EC), 1)
    is_mean = (c_iota % 4) < 2
    sp = jnp.maximum(outs, 0.0) + jnp.log1p(jnp.exp(-jnp.abs(outs)))
    acts = jnp.where(is_mean, jnp.tanh(outs), sp)

    w32 = jax.lax.dot_general(wT, Rep_ref[...], (((0,), (0,)), ((), ())),
                              preferred_element_type=jnp.float32)
    delta = jnp.dot(acts * w32, Csum_ref[...],
                    preferred_element_type=jnp.float32)
    delta_ref[...] = delta
    refined_ref[...] = jnp.clip(c_ref[...] + delta[:, :2] * 0.002, 0.0, 1.0)


def kernel(x, coarse_coord, W1, b1, W2, b2, Ws, bs, Wm, bm, Wv, bv):
    B, S, D = x.shape
    GH = W1.shape[1]
    E = W2.shape[1]
    H = Ws.shape[2]
    N = B * S
    T = 1024
    grid = N // T
    EH = E * H

    x2 = x.reshape(N, D)
    c2 = coarse_coord.reshape(N, 2)
    Wall = jnp.transpose(Ws, (1, 0, 2)).reshape(D, EH)
    Wbig = Wall.astype(jnp.bfloat16)
    bsf = bs.reshape(1, EH)
    Wmv = jnp.concatenate([Wm, Wv], axis=-1)
    eye = jnp.eye(E, dtype=x.dtype)
    Wbd = (eye[:, None, :, None] * Wmv[:, :, None, :]).reshape(EH, E * 4).astype(jnp.bfloat16)
    bmvf = jnp.concatenate([bm, bv], axis=-1).reshape(1, E * 4)
    Rep = jnp.kron(jnp.eye(E, dtype=x.dtype), jnp.ones((1, 4), x.dtype))
    Csum = jnp.tile(jnp.eye(4, dtype=x.dtype), (E, 1))
    b1r = b1.reshape(1, GH)
    b2c = b2.reshape(E, 1)

    body = functools.partial(_moe_kernel, n_tokens=float(N), eh=EH)
    full = lambda i: (0, 0)
    refined, delta, loss = pl.pallas_call(
        body,
        grid=(grid,),
        in_specs=[
            pl.BlockSpec((T, D), lambda i: (i, 0)),
            pl.BlockSpec((T, 2), lambda i: (i, 0)),
            pl.BlockSpec((D, GH), full),
            pl.BlockSpec((1, GH), full),
            pl.BlockSpec((GH, E), full),
            pl.BlockSpec((E, 1), full),
            pl.BlockSpec((D, EH), full),
            pl.BlockSpec((1, EH), full),
            pl.BlockSpec((EH, E * 4), full),
            pl.BlockSpec((1, E * 4), full),
            pl.BlockSpec((E, E * 4), full),
            pl.BlockSpec((E * 4, 4), full),
        ],
        out_specs=[
            pl.BlockSpec((T, 2), lambda i: (i, 0)),
            pl.BlockSpec((T, 4), lambda i: (i, 0)),
            pl.BlockSpec((1, 1), full),
        ],
        out_shape=[
            jax.ShapeDtypeStruct((N, 2), jnp.float32),
            jax.ShapeDtypeStruct((N, 4), jnp.float32),
            jax.ShapeDtypeStruct((1, 1), jnp.float32),
        ],
    )(x2, c2, W1, b1r, W2, b2c, Wbig, bsf, Wbd, bmvf, Rep, Csum)

    return (refined.reshape(B, S, 2), loss[0, 0], delta.reshape(B, S, 4))
